# baseline (speedup 1.0000x reference)
import functools

import jax
import jax.numpy as jnp
from jax import lax
from jax.experimental import pallas as pl
from jax.experimental.pallas import tpu as pltpu
from jax.experimental.pallas import tpu_sc as plsc

D_MODEL = 768
N_TOK = 4 * 8192
NC, NS, L = 2, 16, 16
NW = NC * NS
TOK_W = N_TOK // NW
C = 16
NCH = TOK_W // C
NBUF = 2

_mesh = plsc.VectorSubcoreMesh(core_axis_name="c", subcore_axis_name="s")


@functools.partial(
    pl.kernel,
    out_type=jax.ShapeDtypeStruct((N_TOK, D_MODEL), jnp.float32),
    mesh=_mesh,
    scratch_types=(
        [pltpu.VMEM((NCH, C), jnp.int32)]
        + [pltpu.VMEM((C, D_MODEL), jnp.float32) for _ in range(3 * NBUF)]
        + [pltpu.SemaphoreType.DMA for _ in range(3 * NBUF)]
    ),
)
def _pe_kernel(x_hbm, pos_hbm, tbl_hbm, out_hbm,
               idx_v, xb0, xb1, rb0, rb1, ob0, ob1,
               sx0, sx1, sr0, sr1, so0, so1):
    cid = lax.axis_index("c")
    sid = lax.axis_index("s")
    wid = sid * NC + cid
    base = wid * TOK_W

    xbs, rbs, obs = (xb0, xb1), (rb0, rb1), (ob0, ob1)
    sxs, srs, sos = (sx0, sx1), (sr0, sr1), (so0, so1)

    def fire_x(c, b):
        pltpu.async_copy(x_hbm.at[pl.ds(base + c * C, C)], xbs[b], sxs[b])

    def fire_gather(c, b):
        pltpu.async_copy(tbl_hbm.at[idx_v.at[c]], rbs[b], srs[b])

    fire_x(0, 0)
    fire_x(1, 1)
    pltpu.sync_copy(pos_hbm.at[wid], idx_v)
    fire_gather(0, 0)
    fire_gather(1, 1)

    def outer(g2, carry):
        for b in range(NBUF):
            c = 2 * g2 + b
            tok = base + c * C
            pltpu.make_async_copy(x_hbm.at[pl.ds(0, C)], xbs[b], sxs[b]).wait()
            pltpu.make_async_copy(x_hbm.at[pl.ds(0, C)], rbs[b], srs[b]).wait()

            @pl.when(c >= NBUF)
            def _():
                pltpu.make_async_copy(
                    x_hbm.at[pl.ds(0, C)], obs[b], sos[b]).wait()

            def add_row(t, acc):
                for j in range(D_MODEL // L):
                    sl = pl.ds(j * L, L)
                    obs[b][t, sl] = xbs[b][t, sl] + rbs[b][t, sl]
                return acc

            lax.fori_loop(0, C, add_row, 0)

            pltpu.async_copy(obs[b], out_hbm.at[pl.ds(tok, C)], sos[b])

            @pl.when(c + NBUF < NCH)
            def _():
                fire_x(c + NBUF, b)
                fire_gather(c + NBUF, b)
        return carry

    lax.fori_loop(0, NCH // NBUF, outer, 0)

    for b in range(NBUF):
        pltpu.make_async_copy(x_hbm.at[pl.ds(0, C)], obs[b], sos[b]).wait()


def kernel(x, pos, pos_embedding):
    x2 = x.reshape(N_TOK, D_MODEL)
    idx = pos.astype(jnp.int32).reshape(NW, NCH, C)
    out = _pe_kernel(x2, idx, pos_embedding)
    return out.reshape(x.shape)
